# baseline (speedup 1.0000x reference)
import functools

import jax
import jax.numpy as jnp
from jax import lax
from jax.experimental import pallas as pl
from jax.experimental.pallas import tpu as pltpu
from jax.experimental.pallas import tpu_sc as plsc

_SHAPE_OUT = (256, 256, 64, 10)
_TOTAL_IN = 256 * 256 * 64 * 5
_TOTAL_OUT = 2 * _TOTAL_IN

_NC = 2
_NS = 16
_NW = _NC * _NS

_IN_PER_W = _TOTAL_IN // _NW
_CHUNK_IN = 16 * 1024
_NCHUNK = _IN_PER_W // _CHUNK_IN
_VR_IN = _CHUNK_IN // 16
_UNROLL = 8

_mesh = plsc.VectorSubcoreMesh(core_axis_name="c", subcore_axis_name="s")


@functools.partial(
    pl.kernel,
    out_type=jax.ShapeDtypeStruct((_TOTAL_OUT,), jnp.float32),
    mesh=_mesh,
    scratch_types=[
        pltpu.VMEM((_CHUNK_IN,), jnp.float32),
        pltpu.VMEM((2 * _CHUNK_IN,), jnp.float32),
    ],
)
def _repeat2(in_hbm, out_hbm, in_v, out_v):
    wid = lax.axis_index("s") * _NC + lax.axis_index("c")
    base = wid * _IN_PER_W
    lo_idx = lax.iota(jnp.int32, 16) >> 1
    hi_idx = lo_idx + 8

    def chunk_body(c, _):
        start = base + c * _CHUNK_IN
        pltpu.sync_copy(in_hbm.at[pl.ds(start, _CHUNK_IN)], in_v)

        def vec_body(k, _):
            for u in range(_UNROLL):
                off = (k * _UNROLL + u) * 16
                v = in_v[pl.ds(off, 16)]
                out_v[pl.ds(2 * off, 16)] = v.at[lo_idx].get(
                    mode="promise_in_bounds")
                out_v[pl.ds(2 * off + 16, 16)] = v.at[hi_idx].get(
                    mode="promise_in_bounds")
            return ()

        lax.fori_loop(0, _VR_IN // _UNROLL, vec_body, ())
        pltpu.sync_copy(out_v, out_hbm.at[pl.ds(2 * start, 2 * _CHUNK_IN)])
        return ()

    lax.fori_loop(0, _NCHUNK, chunk_body, ())


def kernel(body):
    flat = body.reshape(_TOTAL_IN)
    return _repeat2(flat).reshape(_SHAPE_OUT)


# speedup vs baseline: 51.0809x; 51.0809x over previous
import functools

import jax
import jax.numpy as jnp
from jax import lax
from jax.experimental import pallas as pl
from jax.experimental.pallas import tpu as pltpu
from jax.experimental.pallas import tpu_sc as plsc

_N = 256
_CIN = 5
_COUT = 10
_SLAB = (64, 256)

_BLOCKS_IN = _N * _CIN
_BLOCKS_OUT = _N * _COUT

_NC = 2
_NS = 16
_NW = _NC * _NS
_IN_PER_W = _BLOCKS_IN // _NW
_NBUF = 2

_mesh = plsc.VectorSubcoreMesh(core_axis_name="c", subcore_axis_name="s")


@functools.partial(
    pl.kernel,
    out_type=jax.ShapeDtypeStruct((_BLOCKS_OUT,) + _SLAB, jnp.float32),
    mesh=_mesh,
    scratch_types=[
        pltpu.VMEM((_NBUF,) + _SLAB, jnp.float32),
        pltpu.SemaphoreType.DMA((_NBUF,)),
        pltpu.SemaphoreType.DMA((_NBUF,)),
    ],
    compiler_params=pltpu.CompilerParams(use_tc_tiling_on_sc=True),
)
def _dup_slabs(in_hbm, out_hbm, buf, in_sem, out_sem):
    wid = lax.axis_index("s") * _NC + lax.axis_index("c")
    base = wid * _IN_PER_W

    def fetch(s, slot):
        pltpu.async_copy(in_hbm.at[base + s], buf.at[slot], in_sem.at[slot])

    def in_wait(slot):
        pltpu.make_async_copy(in_hbm.at[base], buf.at[slot],
                              in_sem.at[slot]).wait()

    def put(s, slot):
        pltpu.async_copy(buf.at[slot], out_hbm.at[2 * (base + s)],
                         out_sem.at[slot])
        pltpu.async_copy(buf.at[slot], out_hbm.at[2 * (base + s) + 1],
                         out_sem.at[slot])

    def out_wait(slot):
        pltpu.make_async_copy(buf.at[slot], out_hbm.at[0],
                              out_sem.at[slot]).wait()
        pltpu.make_async_copy(buf.at[slot], out_hbm.at[0],
                              out_sem.at[slot]).wait()

    fetch(0, 0)

    def step(s, _):
        slot = lax.rem(s, _NBUF)
        nxt = lax.rem(s + 1, _NBUF)

        @pl.when(s + 1 < _IN_PER_W)
        def _():
            @pl.when(s + 1 >= _NBUF)
            def _():
                out_wait(nxt)
            fetch(s + 1, nxt)

        in_wait(slot)
        put(s, slot)
        return ()

    lax.fori_loop(0, _IN_PER_W, step, ())
    out_wait(lax.rem(_IN_PER_W - 1, _NBUF))
    out_wait(lax.rem(_IN_PER_W - 2, _NBUF))


def kernel(body):
    bt = jnp.transpose(body, (0, 3, 2, 1)).reshape((_BLOCKS_IN,) + _SLAB)
    out_t = _dup_slabs(bt)
    out4 = out_t.reshape(_N, _COUT, _SLAB[0], _SLAB[1])
    return jnp.transpose(out4, (0, 3, 2, 1))


# speedup vs baseline: 51.3308x; 1.0049x over previous
import functools

import jax
import jax.numpy as jnp
from jax import lax
from jax.experimental import pallas as pl
from jax.experimental.pallas import tpu as pltpu
from jax.experimental.pallas import tpu_sc as plsc

_N = 256
_CIN = 5
_COUT = 10
_SLAB = (64, 256)

_BLOCKS_IN = _N * _CIN
_BLOCKS_OUT = _N * _COUT

_NC = 2
_NS = 16
_NW = _NC * _NS
_IN_PER_W = _BLOCKS_IN // _NW
_NBUF = 4

_mesh = plsc.VectorSubcoreMesh(core_axis_name="c", subcore_axis_name="s")


@functools.partial(
    pl.kernel,
    out_type=jax.ShapeDtypeStruct((_BLOCKS_OUT,) + _SLAB, jnp.float32),
    mesh=_mesh,
    scratch_types=[
        pltpu.VMEM((_NBUF,) + _SLAB, jnp.float32),
        pltpu.SemaphoreType.DMA((_NBUF,)),
        pltpu.SemaphoreType.DMA((_NBUF,)),
    ],
    compiler_params=pltpu.CompilerParams(use_tc_tiling_on_sc=True),
)
def _dup_slabs(in_hbm, out_hbm, buf, in_sem, out_sem):
    wid = lax.axis_index("s") * _NC + lax.axis_index("c")
    base = wid * _IN_PER_W

    def fetch(s, slot):
        pltpu.async_copy(in_hbm.at[base + s], buf.at[slot], in_sem.at[slot])

    def in_wait(slot):
        pltpu.make_async_copy(in_hbm.at[base], buf.at[slot],
                              in_sem.at[slot]).wait()

    def put(s, slot):
        pltpu.async_copy(buf.at[slot], out_hbm.at[2 * (base + s)],
                         out_sem.at[slot])
        pltpu.async_copy(buf.at[slot], out_hbm.at[2 * (base + s) + 1],
                         out_sem.at[slot])

    def out_wait(slot):
        pltpu.make_async_copy(buf.at[slot], out_hbm.at[0],
                              out_sem.at[slot]).wait()
        pltpu.make_async_copy(buf.at[slot], out_hbm.at[0],
                              out_sem.at[slot]).wait()

    for b in range(_NBUF - 1):
        fetch(b, b)

    def step(s, _):
        slot = lax.rem(s, _NBUF)
        ahead = s + _NBUF - 1

        @pl.when(ahead < _IN_PER_W)
        def _():
            @pl.when(s >= 1)
            def _():
                out_wait(lax.rem(s - 1, _NBUF))
            fetch(ahead, lax.rem(ahead, _NBUF))

        in_wait(slot)
        put(s, slot)
        return ()

    lax.fori_loop(0, _IN_PER_W, step, ())
    for b in range(_NBUF):
        out_wait(b)


def kernel(body):
    bt = jnp.transpose(body, (0, 3, 2, 1)).reshape((_BLOCKS_IN,) + _SLAB)
    out_t = _dup_slabs(bt)
    out4 = out_t.reshape(_N, _COUT, _SLAB[0], _SLAB[1])
    return jnp.transpose(out4, (0, 3, 2, 1))


# speedup vs baseline: 51.9057x; 1.0112x over previous
import functools

import jax
import jax.numpy as jnp
from jax import lax
from jax.experimental import pallas as pl
from jax.experimental.pallas import tpu as pltpu
from jax.experimental.pallas import tpu_sc as plsc

_N = 256
_CIN = 5
_COUT = 10
_SLAB = (64, 256)

_BLOCKS_IN = _N * _CIN
_BLOCKS_OUT = _N * _COUT
_PAIRS = _BLOCKS_IN // 2

_NC = 2
_NS = 16
_NW = _NC * _NS
_PAIRS_PER_W = _PAIRS // _NW
_NBUF = 3

_mesh = plsc.VectorSubcoreMesh(core_axis_name="c", subcore_axis_name="s")


@functools.partial(
    pl.kernel,
    out_type=jax.ShapeDtypeStruct((_BLOCKS_OUT,) + _SLAB, jnp.float32),
    mesh=_mesh,
    scratch_types=[
        pltpu.VMEM((_NBUF, 2) + _SLAB, jnp.float32),
        pltpu.SemaphoreType.DMA((_NBUF,)),
        pltpu.SemaphoreType.DMA((_NBUF,)),
    ],
    compiler_params=pltpu.CompilerParams(use_tc_tiling_on_sc=True),
)
def _dup_slabs(in_hbm, out_hbm, buf, in_sem, out_sem):
    wid = lax.axis_index("s") * _NC + lax.axis_index("c")
    base = wid * _PAIRS_PER_W

    def fetch(p, slot):
        pltpu.async_copy(in_hbm.at[pl.ds(2 * (base + p), 2)], buf.at[slot],
                         in_sem.at[slot])

    def in_wait(slot):
        pltpu.make_async_copy(in_hbm.at[pl.ds(0, 2)], buf.at[slot],
                              in_sem.at[slot]).wait()

    def put(p, slot):
        o = 4 * (base + p)
        pltpu.async_copy(buf.at[slot, 0], out_hbm.at[o], out_sem.at[slot])
        pltpu.async_copy(buf.at[slot], out_hbm.at[pl.ds(o + 1, 2)],
                         out_sem.at[slot])
        pltpu.async_copy(buf.at[slot, 1], out_hbm.at[o + 3], out_sem.at[slot])

    def out_wait(slot):
        pltpu.make_async_copy(buf.at[slot], out_hbm.at[pl.ds(0, 2)],
                              out_sem.at[slot]).wait()
        pltpu.make_async_copy(buf.at[slot, 0], out_hbm.at[0],
                              out_sem.at[slot]).wait()
        pltpu.make_async_copy(buf.at[slot, 1], out_hbm.at[0],
                              out_sem.at[slot]).wait()

    for b in range(_NBUF - 1):
        fetch(b, b)

    def step(p, _):
        slot = lax.rem(p, _NBUF)
        ahead = p + _NBUF - 1

        @pl.when(ahead < _PAIRS_PER_W)
        def _():
            @pl.when(p >= 1)
            def _():
                out_wait(lax.rem(p - 1, _NBUF))
            fetch(ahead, lax.rem(ahead, _NBUF))

        in_wait(slot)
        put(p, slot)
        return ()

    lax.fori_loop(0, _PAIRS_PER_W, step, ())
    for b in range(_NBUF):
        out_wait(b)


def kernel(body):
    bt = jnp.transpose(body, (0, 3, 2, 1)).reshape((_BLOCKS_IN,) + _SLAB)
    out_t = _dup_slabs(bt)
    out4 = out_t.reshape(_N, _COUT, _SLAB[0], _SLAB[1])
    return jnp.transpose(out4, (0, 3, 2, 1))


# speedup vs baseline: 52.5363x; 1.0122x over previous
import functools

import jax
import jax.numpy as jnp
from jax import lax
from jax.experimental import pallas as pl
from jax.experimental.pallas import tpu as pltpu
from jax.experimental.pallas import tpu_sc as plsc

_N = 256
_CIN = 5
_COUT = 10
_SLAB = (64, 256)

_BLOCKS_IN = _N * _CIN
_BLOCKS_OUT = _N * _COUT
_PAIRS = _BLOCKS_IN // 2

_NC = 2
_NS = 16
_NW = _NC * _NS
_PAIRS_PER_W = _PAIRS // _NW
_NBUF = 3
_AHEAD = 1

_mesh = plsc.VectorSubcoreMesh(core_axis_name="c", subcore_axis_name="s")


@functools.partial(
    pl.kernel,
    out_type=jax.ShapeDtypeStruct((_BLOCKS_OUT,) + _SLAB, jnp.float32),
    mesh=_mesh,
    scratch_types=[
        pltpu.VMEM((_NBUF, 2) + _SLAB, jnp.float32),
        pltpu.SemaphoreType.DMA((_NBUF,)),
        pltpu.SemaphoreType.DMA((_NBUF,)),
    ],
    compiler_params=pltpu.CompilerParams(use_tc_tiling_on_sc=True),
)
def _dup_slabs(in_hbm, out_hbm, buf, in_sem, out_sem):
    wid = lax.axis_index("s") * _NC + lax.axis_index("c")
    base = wid * _PAIRS_PER_W

    def fetch(p, slot):
        pltpu.async_copy(in_hbm.at[pl.ds(2 * (base + p), 2)], buf.at[slot],
                         in_sem.at[slot])

    def in_wait(slot):
        pltpu.make_async_copy(in_hbm.at[pl.ds(0, 2)], buf.at[slot],
                              in_sem.at[slot]).wait()

    def put(p, slot):
        o = 4 * (base + p)
        pltpu.async_copy(buf.at[slot, 0], out_hbm.at[o], out_sem.at[slot])
        pltpu.async_copy(buf.at[slot], out_hbm.at[pl.ds(o + 1, 2)],
                         out_sem.at[slot])
        pltpu.async_copy(buf.at[slot, 1], out_hbm.at[o + 3], out_sem.at[slot])

    def out_wait(slot):
        pltpu.make_async_copy(buf.at[slot], out_hbm.at[pl.ds(0, 2)],
                              out_sem.at[slot]).wait()
        pltpu.make_async_copy(buf.at[slot, 0], out_hbm.at[0],
                              out_sem.at[slot]).wait()
        pltpu.make_async_copy(buf.at[slot, 1], out_hbm.at[0],
                              out_sem.at[slot]).wait()

    for b in range(_AHEAD):
        fetch(b, b)

    def step(p, _):
        slot = lax.rem(p, _NBUF)
        ahead = p + _AHEAD

        @pl.when(ahead < _PAIRS_PER_W)
        def _():
            @pl.when(ahead >= _NBUF)
            def _():
                out_wait(lax.rem(ahead, _NBUF))
            fetch(ahead, lax.rem(ahead, _NBUF))

        in_wait(slot)
        put(p, slot)
        return ()

    lax.fori_loop(0, _PAIRS_PER_W, step, ())
    for b in range(_NBUF):
        out_wait(b)


def kernel(body):
    bt = jnp.transpose(body, (0, 3, 2, 1)).reshape((_BLOCKS_IN,) + _SLAB)
    out_t = _dup_slabs(bt)
    out4 = out_t.reshape(_N, _COUT, _SLAB[0], _SLAB[1])
    return jnp.transpose(out4, (0, 3, 2, 1))
